# baseline (speedup 1.0000x reference)
import functools

import jax
import jax.numpy as jnp
from jax import lax
from jax.experimental import pallas as pl
from jax.experimental.pallas import tpu as pltpu
from jax.experimental.pallas import tpu_sc as plsc

_NUM_CORES = 2
_NUM_SUBCORES = 16
_NUM_WORKERS = _NUM_CORES * _NUM_SUBCORES
_LANES = 16
_IDX_CHUNK = 128


def _build_sc_call(B, D, Vu, Vm):
    b_per_w = B // _NUM_WORKERS
    n_chunks = b_per_w // _IDX_CHUNK
    n_groups = b_per_w // _LANES

    mesh = plsc.VectorSubcoreMesh(core_axis_name="c", subcore_axis_name="s")

    @functools.partial(
        pl.kernel,
        mesh=mesh,
        compiler_params=pltpu.CompilerParams(
            needs_layout_passes=False, use_tc_tiling_on_sc=False),
        out_type=jax.ShapeDtypeStruct((B,), jnp.float32),
        scratch_types=[
            pltpu.VMEM((n_chunks, _IDX_CHUNK), jnp.int32),
            pltpu.VMEM((n_chunks, _IDX_CHUNK), jnp.int32),
            pltpu.VMEM((b_per_w, D), jnp.float32),
            pltpu.VMEM((b_per_w, D), jnp.float32),
            pltpu.VMEM((2 * D + 8,), jnp.float32),
            pltpu.VMEM((b_per_w,), jnp.float32),
            pltpu.SemaphoreType.DMA,
        ],
    )
    def sc_call(users_hbm, movies_hbm, ut_hbm, mt_hbm, wb_hbm, out_hbm,
                uidx_v, midx_v, urows_v, mrows_v, w_v, out_v, sem):
        wid = lax.axis_index("s") * _NUM_CORES + lax.axis_index("c")
        base = wid * b_per_w

        pltpu.sync_copy(wb_hbm, w_v)
        for j in range(n_chunks):
            off = base + j * _IDX_CHUNK
            pltpu.sync_copy(users_hbm.at[pl.ds(off, _IDX_CHUNK)], uidx_v.at[j])
            pltpu.sync_copy(movies_hbm.at[pl.ds(off, _IDX_CHUNK)], midx_v.at[j])

        copies = []
        for j in range(n_chunks):
            dst = pl.ds(j * _IDX_CHUNK, _IDX_CHUNK)
            copies.append(pltpu.async_copy(ut_hbm.at[uidx_v.at[j]], urows_v.at[dst], sem))
            copies.append(pltpu.async_copy(mt_hbm.at[midx_v.at[j]], mrows_v.at[dst], sem))
        for c in copies:
            c.wait()

        lane = lax.iota(jnp.int32, _LANES)
        w0 = w_v[pl.ds(0, _LANES)]
        w1 = w_v[pl.ds(_LANES, _LANES)]
        w2 = w_v[pl.ds(2 * _LANES, _LANES)]
        w3 = w_v[pl.ds(3 * _LANES, _LANES)]
        bias_vec = plsc.load_gather(w_v, [jnp.full((_LANES,), 2 * D, jnp.int32)])
        last = lane == (_LANES - 1)
        lane_hi = lane + _LANES

        def row_body(i, carry):
            r = jnp.full((_LANES,), i, jnp.int32)
            u0 = plsc.load_gather(urows_v, [r, lane])
            u1 = plsc.load_gather(urows_v, [r, lane_hi])
            m0 = plsc.load_gather(mrows_v, [r, lane])
            m1 = plsc.load_gather(mrows_v, [r, lane_hi])
            t = u0 * w0 + u1 * w1 + m0 * w2 + m1 * w3
            s = plsc.cumsum(t) + bias_vec
            plsc.store_scatter(out_v, [r], s, mask=last)
            return carry

        lax.fori_loop(0, b_per_w, row_body, 0)

        pltpu.sync_copy(out_v, out_hbm.at[pl.ds(base, b_per_w)])

    return sc_call


def kernel(users, movies, user_table, movie_table, W, b):
    B = users.shape[0]
    D = user_table.shape[1]
    wb = jnp.concatenate([
        W.reshape(-1).astype(jnp.float32),
        b.reshape(-1).astype(jnp.float32),
        jnp.zeros((7,), jnp.float32),
    ])
    sc_call = _build_sc_call(B, D, user_table.shape[0], movie_table.shape[0])
    out = sc_call(users.astype(jnp.int32), movies.astype(jnp.int32),
                  user_table, movie_table, wb)
    return out.reshape(B, 1)
